# baseline (speedup 1.0000x reference)
import functools

import jax
import jax.numpy as jnp
from jax import lax
from jax.experimental import pallas as pl
from jax.experimental.pallas import tpu as pltpu
from jax.experimental.pallas import tpu_sc as plsc

HIDDEN = 128
MAX_STEPS = 20
NUM_GRAPHS = 100
NPG = 1000
LANES = 16
NC = 2
NS = 16
NW = NC * NS
VREGS = HIDDEN // LANES
NGROUPS = 63
PAD_MASK = 1024
GRAPHS_PER_W = (NUM_GRAPHS + NW - 1) // NW

CH_ROWS = 256
LAST_ROWS = NPG - 3 * CH_ROWS
NCH = 4


def _lane_splat(v, r):
    idx = jnp.full((LANES, 1), r, dtype=jnp.int32)
    dnums = lax.GatherDimensionNumbers(
        offset_dims=(), collapsed_slice_dims=(0,), start_index_map=(0,))
    return lax.gather(v, idx, dnums, slice_sizes=(1,),
                      mode=lax.GatherScatterMode.PROMISE_IN_BOUNDS)


LAST_K = GRAPHS_PER_W - 1
W4 = NUM_GRAPHS - NW * LAST_K


def _sc_body(nodes_hbm, maskf_hbm, sums_hbm, cnts_hbm,
             xbuf_a, xbuf_b, xbuf_c, mbuf_a, mbuf_b, ostage, ostage_c,
             sx_a, sx_b, sx_c, sm_a, sm_b, so, soc):
    wid = lax.axis_index("s") * NC + lax.axis_index("c")
    xb = (xbuf_a, xbuf_b, xbuf_c)
    xs = (sx_a, sx_b, sx_c)
    mb = (mbuf_a, mbuf_b)
    ms = (sm_a, sm_b)

    zf = jnp.zeros((LANES,), jnp.float32)
    HALF = VREGS // 2

    iota_f = lax.broadcasted_iota(jnp.int32, (LANES,), 0).astype(jnp.float32)
    lowmask = jnp.clip(2.0 - iota_f, 0.0, 1.0)

    def make_pass_body(x_ref, m_ref, moff, jlo, with_cnt):
        def pass_body(i, carry):
            base = i * 2
            accs = list(carry[:HALF])
            m16 = m_ref[pl.ds(moff + base, LANES)]
            if with_cnt:
                cnt = carry[HALF] + m16 * lowmask
            for r in range(2):
                msp = _lane_splat(m16, r)
                row = base + r
                for j in range(jlo, jlo + HALF):
                    accs[j - jlo] = (accs[j - jlo]
                                     + msp * x_ref[row, pl.ds(j * LANES, LANES)])
            if with_cnt:
                return tuple(accs) + (cnt,)
            return tuple(accs)
        return pass_body

    def chunk_accum(x_ref, m_ref, moff, ngroups, carry):
        lo = lax.fori_loop(0, ngroups,
                           make_pass_body(x_ref, m_ref, moff, 0, True),
                           tuple(carry[:HALF]) + (carry[VREGS],))
        hi = lax.fori_loop(0, ngroups,
                           make_pass_body(x_ref, m_ref, moff, HALF, False),
                           tuple(carry[HALF:VREGS]))
        return tuple(lo[:HALF]) + tuple(hi) + (lo[HALF],)

    def fire_x(idx):
        k, c = divmod(idx, NCH)
        row0 = (wid + NW * k) * NPG + c * CH_ROWS
        if c < NCH - 1:
            return pltpu.async_copy(nodes_hbm.at[pl.ds(row0, CH_ROWS)],
                                    xb[idx % 3], xs[idx % 3])
        return pltpu.async_copy(nodes_hbm.at[pl.ds(row0, LAST_ROWS)],
                                xb[idx % 3].at[pl.ds(0, LAST_ROWS)],
                                xs[idx % 3])

    def fire_m(k):
        return pltpu.async_copy(maskf_hbm.at[wid + NW * k],
                                mb[k % 2], ms[k % 2])

    def finish_graph(k, carry):
        g = wid + NW * k
        for j in range(VREGS):
            ostage[k, pl.ds(j * LANES, LANES)] = carry[j]
        ostage_c[k, pl.ds(0, LANES)] = carry[VREGS]
        o1 = pltpu.async_copy(ostage.at[k], sums_hbm.at[g], so)
        o2 = pltpu.async_copy(ostage_c.at[k], cnts_hbm.at[g], soc)
        return o1, o2

    GROUPS = (128, 128, 128, 116)

    mcs = {0: fire_m(0)}
    cps = {0: fire_x(0), 1: fire_x(1), 2: fire_x(2)}
    ocps = []
    for k in range(LAST_K):
        if k + 1 < LAST_K:
            mcs[k + 1] = fire_m(k + 1)
        mcs[k].wait()
        carry = tuple(zf for _ in range(VREGS + 1))
        for c in range(NCH):
            idx = NCH * k + c
            cps[idx].wait()
            carry = chunk_accum(xb[idx % 3], mb[k % 2], c * CH_ROWS,
                                GROUPS[c], carry)
            if idx + 3 < NCH * LAST_K:
                cps[idx + 3] = fire_x(idx + 3)
        ocps.extend(finish_graph(k, carry))

    @pl.when(wid < W4)
    def _():
        k = LAST_K
        m3 = fire_m(k)
        c3 = [fire_x(NCH * k), fire_x(NCH * k + 1), fire_x(NCH * k + 2)]
        m3.wait()
        carry = tuple(zf for _ in range(VREGS + 1))
        for c in range(NCH):
            idx = NCH * k + c
            if c < NCH - 1:
                cp = c3[c]
            else:
                cp = c3[NCH - 1]
            cp.wait()
            carry = chunk_accum(xb[idx % 3], mb[k % 2], c * CH_ROWS,
                                GROUPS[c], carry)
            if c == 0:
                c3.append(fire_x(NCH * k + 3))
        o1, o2 = finish_graph(k, carry)
        o1.wait()
        o2.wait()

    for o in ocps:
        o.wait()


_sc_segsum = functools.partial(
    pl.kernel,
    mesh=plsc.VectorSubcoreMesh(core_axis_name="c", subcore_axis_name="s",
                                num_cores=NC, num_subcores=NS),
    out_type=(
        jax.ShapeDtypeStruct((NUM_GRAPHS, HIDDEN), jnp.float32),
        jax.ShapeDtypeStruct((NUM_GRAPHS, LANES), jnp.float32),
    ),
    scratch_types=(
        pltpu.VMEM((CH_ROWS, HIDDEN), jnp.float32),
        pltpu.VMEM((CH_ROWS, HIDDEN), jnp.float32),
        pltpu.VMEM((CH_ROWS, HIDDEN), jnp.float32),
        pltpu.VMEM((PAD_MASK,), jnp.float32),
        pltpu.VMEM((PAD_MASK,), jnp.float32),
        pltpu.VMEM((GRAPHS_PER_W, HIDDEN), jnp.float32),
        pltpu.VMEM((GRAPHS_PER_W, LANES), jnp.float32),
        pltpu.SemaphoreType.DMA,
        pltpu.SemaphoreType.DMA,
        pltpu.SemaphoreType.DMA,
        pltpu.SemaphoreType.DMA,
        pltpu.SemaphoreType.DMA,
        pltpu.SemaphoreType.DMA,
        pltpu.SemaphoreType.DMA,
    ),
)(_sc_body)


def _combine_body(sums_ref, cnts_ref, q_ref, emb_ref, sc_ref, gam_ref,
                  bet_ref, o_ref):
    cnt = jnp.maximum(jnp.sum(cnts_ref[...], axis=1, keepdims=True), 1.0)
    mean = sums_ref[...] / cnt
    sc = jnp.clip(sc_ref[...].astype(jnp.float32), 0.0, float(MAX_STEPS))
    rem = float(MAX_STEPS) - sc
    iota = lax.broadcasted_iota(
        jnp.int32, (NUM_GRAPHS, MAX_STEPS + 1), 1).astype(jnp.float32)
    d = iota - rem
    oh = jnp.maximum(1.0 - d * d, 0.0)
    emb = jnp.dot(oh, emb_ref[...], preferred_element_type=jnp.float32)
    st = mean + q_ref[...] + emb
    mu = jnp.mean(st, axis=1, keepdims=True)
    var = jnp.mean((st - mu) ** 2, axis=1, keepdims=True)
    o_ref[...] = ((st - mu) * lax.rsqrt(var + 1e-5) * gam_ref[...]
                  + bet_ref[...])


def kernel(node_tokens, question_tokens, step_emb_weight, ln_gamma, ln_beta,
           node_ptr, active_nodes, step_counts):
    del node_ptr
    maskf = jnp.pad(active_nodes.astype(jnp.float32).reshape(NUM_GRAPHS, NPG),
                    ((0, 0), (0, PAD_MASK - NPG)))
    sums, cnts = _sc_segsum(node_tokens, maskf)
    sc2d = step_counts.astype(jnp.int32).reshape(NUM_GRAPHS, 1)
    out = pl.pallas_call(
        _combine_body,
        out_shape=jax.ShapeDtypeStruct((NUM_GRAPHS, HIDDEN), jnp.float32),
    )(sums, cnts, question_tokens, step_emb_weight, sc2d,
      ln_gamma.reshape(1, HIDDEN), ln_beta.reshape(1, HIDDEN))
    return out


# speedup vs baseline: 1.0084x; 1.0084x over previous
import functools

import jax
import jax.numpy as jnp
from jax import lax
from jax.experimental import pallas as pl
from jax.experimental.pallas import tpu as pltpu
from jax.experimental.pallas import tpu_sc as plsc

HIDDEN = 128
MAX_STEPS = 20
NUM_GRAPHS = 100
NPG = 1000
LANES = 16
NC = 2
NS = 16
NW = NC * NS
VREGS = HIDDEN // LANES
NGROUPS = 63
PAD_MASK = 1024
GRAPHS_PER_W = (NUM_GRAPHS + NW - 1) // NW

CH_ROWS = 256
LAST_ROWS = NPG - 3 * CH_ROWS
NCH = 4


def _lane_splat(v, r):
    idx = jnp.full((LANES, 1), r, dtype=jnp.int32)
    dnums = lax.GatherDimensionNumbers(
        offset_dims=(), collapsed_slice_dims=(0,), start_index_map=(0,))
    return lax.gather(v, idx, dnums, slice_sizes=(1,),
                      mode=lax.GatherScatterMode.PROMISE_IN_BOUNDS)


LAST_K = GRAPHS_PER_W - 1
W4 = NUM_GRAPHS - NW * LAST_K


def _sc_body(nodes_hbm, maskf_hbm, sums_hbm, cnts_hbm,
             xbuf_a, xbuf_b, xbuf_c, mbuf_a, mbuf_b, ostage, ostage_c,
             sx_a, sx_b, sx_c, sm_a, sm_b, so, soc):
    wid = lax.axis_index("s") * NC + lax.axis_index("c")
    xb = (xbuf_a, xbuf_b, xbuf_c)
    xs = (sx_a, sx_b, sx_c)
    mb = (mbuf_a, mbuf_b)
    ms = (sm_a, sm_b)

    zf = jnp.zeros((LANES,), jnp.float32)
    HALF = VREGS // 2

    iota_f = lax.broadcasted_iota(jnp.int32, (LANES,), 0).astype(jnp.float32)
    lowmask = jnp.clip(4.0 - iota_f, 0.0, 1.0)

    def make_pass_body(x_ref, m_ref, moff, jlo, with_cnt):
        def pass_body(i, carry):
            base = i * 4
            accs = list(carry[:HALF])
            m16 = m_ref[pl.ds(moff + base, LANES)]
            if with_cnt:
                cnt = carry[HALF] + m16 * lowmask
            for r in range(4):
                msp = _lane_splat(m16, r)
                row = base + r
                for j in range(jlo, jlo + HALF):
                    accs[j - jlo] = (accs[j - jlo]
                                     + msp * x_ref[row, pl.ds(j * LANES, LANES)])
            if with_cnt:
                return tuple(accs) + (cnt,)
            return tuple(accs)
        return pass_body

    def make_full_body(x_ref, m_ref, moff):
        def body(i, carry):
            base = i * 4
            accs = list(carry[:VREGS])
            m16 = m_ref[pl.ds(moff + base, LANES)]
            cnt = carry[VREGS] + m16 * lowmask
            for r in range(4):
                msp = _lane_splat(m16, r)
                row = base + r
                for j in range(VREGS):
                    accs[j] = (accs[j]
                               + msp * x_ref[row, pl.ds(j * LANES, LANES)])
            return tuple(accs) + (cnt,)
        return body

    def chunk_accum(x_ref, m_ref, moff, ngroups, carry):
        return lax.fori_loop(0, ngroups, make_full_body(x_ref, m_ref, moff),
                             carry)

    def fire_x(idx):
        k, c = divmod(idx, NCH)
        row0 = (wid + NW * k) * NPG + c * CH_ROWS
        if c < NCH - 1:
            return pltpu.async_copy(nodes_hbm.at[pl.ds(row0, CH_ROWS)],
                                    xb[idx % 3], xs[idx % 3])
        return pltpu.async_copy(nodes_hbm.at[pl.ds(row0, LAST_ROWS)],
                                xb[idx % 3].at[pl.ds(0, LAST_ROWS)],
                                xs[idx % 3])

    def fire_m(k):
        return pltpu.async_copy(maskf_hbm.at[wid + NW * k],
                                mb[k % 2], ms[k % 2])

    def finish_graph(k, carry):
        g = wid + NW * k
        for j in range(VREGS):
            ostage[k, pl.ds(j * LANES, LANES)] = carry[j]
        ostage_c[k, pl.ds(0, LANES)] = carry[VREGS]
        o1 = pltpu.async_copy(ostage.at[k], sums_hbm.at[g], so)
        o2 = pltpu.async_copy(ostage_c.at[k], cnts_hbm.at[g], soc)
        return o1, o2

    GROUPS = (64, 64, 64, 58)

    mcs = {0: fire_m(0)}
    cps = {0: fire_x(0), 1: fire_x(1), 2: fire_x(2)}
    ocps = []
    for k in range(LAST_K):
        if k + 1 < LAST_K:
            mcs[k + 1] = fire_m(k + 1)
        mcs[k].wait()
        carry = tuple(zf for _ in range(VREGS + 1))
        for c in range(NCH):
            idx = NCH * k + c
            cps[idx].wait()
            carry = chunk_accum(xb[idx % 3], mb[k % 2], c * CH_ROWS,
                                GROUPS[c], carry)
            if idx + 3 < NCH * LAST_K:
                cps[idx + 3] = fire_x(idx + 3)
        ocps.extend(finish_graph(k, carry))

    @pl.when(wid < W4)
    def _():
        k = LAST_K
        m3 = fire_m(k)
        c3 = [fire_x(NCH * k), fire_x(NCH * k + 1), fire_x(NCH * k + 2)]
        m3.wait()
        carry = tuple(zf for _ in range(VREGS + 1))
        for c in range(NCH):
            idx = NCH * k + c
            if c < NCH - 1:
                cp = c3[c]
            else:
                cp = c3[NCH - 1]
            cp.wait()
            carry = chunk_accum(xb[idx % 3], mb[k % 2], c * CH_ROWS,
                                GROUPS[c], carry)
            if c == 0:
                c3.append(fire_x(NCH * k + 3))
        o1, o2 = finish_graph(k, carry)
        o1.wait()
        o2.wait()

    for o in ocps:
        o.wait()


_sc_segsum = functools.partial(
    pl.kernel,
    mesh=plsc.VectorSubcoreMesh(core_axis_name="c", subcore_axis_name="s",
                                num_cores=NC, num_subcores=NS),
    out_type=(
        jax.ShapeDtypeStruct((NUM_GRAPHS, HIDDEN), jnp.float32),
        jax.ShapeDtypeStruct((NUM_GRAPHS, LANES), jnp.float32),
    ),
    scratch_types=(
        pltpu.VMEM((CH_ROWS, HIDDEN), jnp.float32),
        pltpu.VMEM((CH_ROWS, HIDDEN), jnp.float32),
        pltpu.VMEM((CH_ROWS, HIDDEN), jnp.float32),
        pltpu.VMEM((PAD_MASK,), jnp.float32),
        pltpu.VMEM((PAD_MASK,), jnp.float32),
        pltpu.VMEM((GRAPHS_PER_W, HIDDEN), jnp.float32),
        pltpu.VMEM((GRAPHS_PER_W, LANES), jnp.float32),
        pltpu.SemaphoreType.DMA,
        pltpu.SemaphoreType.DMA,
        pltpu.SemaphoreType.DMA,
        pltpu.SemaphoreType.DMA,
        pltpu.SemaphoreType.DMA,
        pltpu.SemaphoreType.DMA,
        pltpu.SemaphoreType.DMA,
    ),
)(_sc_body)


def _combine_body(sums_ref, cnts_ref, q_ref, emb_ref, sc_ref, gam_ref,
                  bet_ref, o_ref):
    cnt = jnp.maximum(jnp.sum(cnts_ref[...], axis=1, keepdims=True), 1.0)
    mean = sums_ref[...] / cnt
    sc = jnp.clip(sc_ref[...].astype(jnp.float32), 0.0, float(MAX_STEPS))
    rem = float(MAX_STEPS) - sc
    iota = lax.broadcasted_iota(
        jnp.int32, (NUM_GRAPHS, MAX_STEPS + 1), 1).astype(jnp.float32)
    d = iota - rem
    oh = jnp.maximum(1.0 - d * d, 0.0)
    emb = jnp.dot(oh, emb_ref[...], preferred_element_type=jnp.float32)
    st = mean + q_ref[...] + emb
    mu = jnp.mean(st, axis=1, keepdims=True)
    var = jnp.mean((st - mu) ** 2, axis=1, keepdims=True)
    o_ref[...] = ((st - mu) * lax.rsqrt(var + 1e-5) * gam_ref[...]
                  + bet_ref[...])


def kernel(node_tokens, question_tokens, step_emb_weight, ln_gamma, ln_beta,
           node_ptr, active_nodes, step_counts):
    del node_ptr
    maskf = jnp.pad(active_nodes.astype(jnp.float32).reshape(NUM_GRAPHS, NPG),
                    ((0, 0), (0, PAD_MASK - NPG)))
    sums, cnts = _sc_segsum(node_tokens, maskf)
    sc2d = step_counts.astype(jnp.int32).reshape(NUM_GRAPHS, 1)
    out = pl.pallas_call(
        _combine_body,
        out_shape=jax.ShapeDtypeStruct((NUM_GRAPHS, HIDDEN), jnp.float32),
    )(sums, cnts, question_tokens, step_emb_weight, sc2d,
      ln_gamma.reshape(1, HIDDEN), ln_beta.reshape(1, HIDDEN))
    return out
